# baseline (speedup 1.0000x reference)
import jax
import jax.numpy as jnp
from jax.experimental import pallas as pl

_HIDDEN = 2048
_EXPERTS = 16
_TOKENS = 16384
_BT = 2048


def _body(x_ref, wt_ref, b_ref, ow_ref, oi_ref):
    logits = (
        jnp.dot(x_ref[:], wt_ref[:], preferred_element_type=jnp.float32)
        + b_ref[:]
    )
    cols = jax.lax.broadcasted_iota(jnp.int32, (_BT, _EXPERTS), 1)
    m1 = jnp.max(logits, axis=1, keepdims=True)
    i1 = jnp.argmax(logits, axis=1).astype(jnp.int32)
    rest = jnp.where(cols == i1[:, None], -jnp.inf, logits)
    m2 = jnp.max(rest, axis=1, keepdims=True)
    i2 = jnp.argmax(rest, axis=1).astype(jnp.int32)
    w1 = 1.0 / (1.0 + jnp.exp(m2 - m1))
    w2 = 1.0 - w1
    ow_ref[:] = jnp.concatenate([w1, w2], axis=1)
    oi_ref[:] = jnp.stack([i1, i2], axis=1)


def kernel(x, W, b):
    wt = W.T
    b2 = b.reshape(1, _EXPERTS)
    return pl.pallas_call(
        _body,
        grid=(_TOKENS // _BT,),
        in_specs=[
            pl.BlockSpec((_BT, _HIDDEN), lambda i: (i, 0)),
            pl.BlockSpec((_HIDDEN, _EXPERTS), lambda i: (0, 0)),
            pl.BlockSpec((1, _EXPERTS), lambda i: (0, 0)),
        ],
        out_specs=[
            pl.BlockSpec((_BT, 2), lambda i: (i, 0)),
            pl.BlockSpec((_BT, 2), lambda i: (i, 0)),
        ],
        out_shape=[
            jax.ShapeDtypeStruct((_TOKENS, 2), jnp.float32),
            jax.ShapeDtypeStruct((_TOKENS, 2), jnp.int32),
        ],
    )(x, wt, b2)


# speedup vs baseline: 1.3521x; 1.3521x over previous
import jax
import jax.numpy as jnp
from jax.experimental import pallas as pl

_HIDDEN = 2048
_EXPERTS = 16
_TOKENS = 16384
_BT = 2048


def _body(x_ref, wt_ref, b_ref, ow_ref, oi_ref):
    logits = (
        jnp.dot(x_ref[:], wt_ref[:], preferred_element_type=jnp.float32)
        + b_ref[:]
    )
    lt = logits.T
    rows = jax.lax.broadcasted_iota(jnp.int32, (_EXPERTS, _BT), 0)
    m1 = jnp.max(lt, axis=0, keepdims=True)
    i1 = jnp.min(
        jnp.where(lt == m1, rows, _EXPERTS), axis=0, keepdims=True
    )
    rest = jnp.where(rows == i1, -jnp.inf, lt)
    m2 = jnp.max(rest, axis=0, keepdims=True)
    i2 = jnp.min(
        jnp.where(rest == m2, rows, _EXPERTS), axis=0, keepdims=True
    )
    w1 = 1.0 / (1.0 + jnp.exp(m2 - m1))
    w2 = 1.0 - w1
    ow_ref[:] = jnp.concatenate([w1, w2], axis=0)
    oi_ref[:] = jnp.concatenate([i1, i2], axis=0)


def kernel(x, W, b):
    wt = W.T
    b2 = b.reshape(1, _EXPERTS)
    w_t, i_t = pl.pallas_call(
        _body,
        grid=(_TOKENS // _BT,),
        in_specs=[
            pl.BlockSpec((_BT, _HIDDEN), lambda i: (i, 0)),
            pl.BlockSpec((_HIDDEN, _EXPERTS), lambda i: (0, 0)),
            pl.BlockSpec((1, _EXPERTS), lambda i: (0, 0)),
        ],
        out_specs=[
            pl.BlockSpec((2, _BT), lambda i: (0, i)),
            pl.BlockSpec((2, _BT), lambda i: (0, i)),
        ],
        out_shape=[
            jax.ShapeDtypeStruct((2, _TOKENS), jnp.float32),
            jax.ShapeDtypeStruct((2, _TOKENS), jnp.int32),
        ],
    )(x, wt, b2)
    return w_t.T, i_t.T


# speedup vs baseline: 1.3908x; 1.0286x over previous
import jax
import jax.numpy as jnp
from jax.experimental import pallas as pl

_HIDDEN = 2048
_EXPERTS = 16
_TOKENS = 16384
_BT = 1024


def _body(x_ref, wt_ref, b_ref, ow_ref, oi_ref):
    logits = (
        jnp.dot(x_ref[:], wt_ref[:], preferred_element_type=jnp.float32)
        + b_ref[:]
    )
    lt = logits.T
    rows = jax.lax.broadcasted_iota(jnp.int32, (_EXPERTS, _BT), 0)
    m1 = jnp.max(lt, axis=0, keepdims=True)
    i1 = jnp.min(
        jnp.where(lt == m1, rows, _EXPERTS), axis=0, keepdims=True
    )
    rest = jnp.where(rows == i1, -jnp.inf, lt)
    m2 = jnp.max(rest, axis=0, keepdims=True)
    i2 = jnp.min(
        jnp.where(rest == m2, rows, _EXPERTS), axis=0, keepdims=True
    )
    w1 = 1.0 / (1.0 + jnp.exp(m2 - m1))
    w2 = 1.0 - w1
    ow_ref[:] = jnp.concatenate([w1, w2], axis=0)
    oi_ref[:] = jnp.concatenate([i1, i2], axis=0)


def kernel(x, W, b):
    wt = W.T
    b2 = b.reshape(1, _EXPERTS)
    w_t, i_t = pl.pallas_call(
        _body,
        grid=(_TOKENS // _BT,),
        in_specs=[
            pl.BlockSpec((_BT, _HIDDEN), lambda i: (i, 0)),
            pl.BlockSpec((_HIDDEN, _EXPERTS), lambda i: (0, 0)),
            pl.BlockSpec((1, _EXPERTS), lambda i: (0, 0)),
        ],
        out_specs=[
            pl.BlockSpec((2, _BT), lambda i: (0, i)),
            pl.BlockSpec((2, _BT), lambda i: (0, i)),
        ],
        out_shape=[
            jax.ShapeDtypeStruct((2, _TOKENS), jnp.float32),
            jax.ShapeDtypeStruct((2, _TOKENS), jnp.int32),
        ],
    )(x, wt, b2)
    return w_t.T, i_t.T


# speedup vs baseline: 1.4886x; 1.0703x over previous
import jax
import jax.numpy as jnp
from jax.experimental import pallas as pl

_HIDDEN = 2048
_EXPERTS = 16
_TOKENS = 16384
_BT = 1024


def _body(x_ref, w_ref, b_ref, ow_ref, oi_ref):
    logits = (
        jax.lax.dot_general(
            x_ref[:],
            w_ref[:],
            dimension_numbers=(((1,), (1,)), ((), ())),
            preferred_element_type=jnp.float32,
        )
        + b_ref[:]
    )
    lt = logits.T
    rows = jax.lax.broadcasted_iota(jnp.int32, (_EXPERTS, _BT), 0)
    m1 = jnp.max(lt, axis=0, keepdims=True)
    i1 = jnp.min(
        jnp.where(lt == m1, rows, _EXPERTS), axis=0, keepdims=True
    )
    rest = jnp.where(rows == i1, -jnp.inf, lt)
    m2 = jnp.max(rest, axis=0, keepdims=True)
    i2 = jnp.min(
        jnp.where(rest == m2, rows, _EXPERTS), axis=0, keepdims=True
    )
    w1 = 1.0 / (1.0 + jnp.exp(m2 - m1))
    w2 = 1.0 - w1
    ow_ref[:] = jnp.concatenate([w1, w2], axis=0)
    oi_ref[:] = jnp.concatenate([i1, i2], axis=0)


def kernel(x, W, b):
    b2 = b.reshape(1, _EXPERTS)
    w_t, i_t = pl.pallas_call(
        _body,
        grid=(_TOKENS // _BT,),
        in_specs=[
            pl.BlockSpec((_BT, _HIDDEN), lambda i: (i, 0)),
            pl.BlockSpec((_EXPERTS, _HIDDEN), lambda i: (0, 0)),
            pl.BlockSpec((1, _EXPERTS), lambda i: (0, 0)),
        ],
        out_specs=[
            pl.BlockSpec((2, _BT), lambda i: (0, i)),
            pl.BlockSpec((2, _BT), lambda i: (0, i)),
        ],
        out_shape=[
            jax.ShapeDtypeStruct((2, _TOKENS), jnp.float32),
            jax.ShapeDtypeStruct((2, _TOKENS), jnp.int32),
        ],
    )(x, W, b2)
    return w_t.T, i_t.T
